# Initial kernel scaffold; baseline (speedup 1.0000x reference)
import jax
import jax.numpy as jnp
from jax.experimental import pallas as pl


def kernel(target_bbox, rpn_match, rpn_bbox):
    raise NotImplementedError("write your pallas kernel here")


# speedup vs baseline: 7.7487x; 7.7487x over previous
import functools

import jax
import jax.numpy as jnp
from jax import lax
from jax.experimental import pallas as pl
from jax.experimental.pallas import tpu as pltpu
from jax.experimental.pallas import tpu_sc as plsc

B = 8
A = 261888
MP = 256
NQ = 4
CH = A // NQ
NV = CH // 16
U = 6
K = 512
GW = 128
NC = 2
NS = 16
NW = NC * NS

_mesh = plsc.VectorSubcoreMesh(
    core_axis_name="c", subcore_axis_name="s", num_cores=NC, num_subcores=NS
)


@functools.partial(
    pl.kernel,
    out_type=jax.ShapeDtypeStruct((NW * 16,), jnp.float32),
    mesh=_mesh,
    scratch_types=[
        pltpu.VMEM((CH,), jnp.int32),
        pltpu.VMEM((K,), jnp.int32),
        pltpu.VMEM((K * 4,), jnp.int32),
        pltpu.VMEM((K * 4,), jnp.float32),
        pltpu.VMEM((MP * 4,), jnp.float32),
        pltpu.VMEM((16,), jnp.float32),
        pltpu.SMEM((32,), jnp.int32),
        pltpu.SemaphoreType.DMA,
    ],
    compiler_params=pltpu.CompilerParams(needs_layout_passes=False),
)
def _rpn_loss_sc(match_hbm, tgt_hbm, rpn_hbm, out_hbm,
                 chunk_v, idx_v, idx4_v, rows_v, tgt_v, sum_v,
                 flag_smem, sem):
    c = lax.axis_index("c")
    s = lax.axis_index("s")
    b = c * 4 + s // 4
    q = s % 4
    img = s // 4
    iota = lax.iota(jnp.int32, 16)

    for w in range(18):
        flag_smem[w] = 0

    pltpu.sync_copy(match_hbm.at[pl.ds(b * A + q * CH, CH)], chunk_v)
    pltpu.sync_copy(tgt_hbm.at[pl.ds(b * (MP * 4), MP * 4)], tgt_v)

    row_base = b * A + q * CH

    def fill_body(i, carry):
        idx_v[pl.ds(i * 16, 16)] = row_base + i * 16 + iota
        return carry

    lax.fori_loop(0, K // 16, fill_body, 0)

    def scan_body(i, off_vec):
        for u in range(U):
            base_e = (i * U + u) * 16
            v = chunk_v[pl.ds(base_e, 16)]
            m = v == 1
            slot = off_vec + plsc.cumsum(m.astype(jnp.int32)) - 1
            slot = jnp.clip(slot, 0, K - 1)
            plsc.store_scatter(idx_v, [slot], row_base + base_e + iota, mask=m)
            off_vec = off_vec + plsc.all_reduce_population_count(m)
        return off_vec

    off_vec = lax.fori_loop(0, NV // U, scan_body, jnp.zeros((16,), jnp.int32))
    cnt = jnp.max(off_vec)

    for qq in range(NQ):
        add = jnp.where(qq > q, cnt, jnp.int32(0))
        plsc.fetch_and_add(
            flag_smem.at[2 + img * 4 + qq], add, subcore_id=jnp.int32(0)
        )
    plsc.fetch_and_add(flag_smem.at[0], jnp.int32(1), subcore_id=jnp.int32(0))

    def spin_cond(seen):
        return seen < NS

    def spin_body(seen):
        return plsc.fetch_and_add(
            flag_smem.at[0], jnp.int32(0), subcore_id=jnp.int32(0)
        )

    lax.while_loop(spin_cond, spin_body, jnp.int32(0))

    base = plsc.fetch_and_add(
        flag_smem.at[2 + img * 4 + q], jnp.int32(0), subcore_id=jnp.int32(0)
    )

    def exp_body(t, carry):
        e = t * 16 + iota
        flat = plsc.load_gather(idx_v, [e // 4])
        idx4_v[pl.ds(t * 16, 16)] = flat * 4 + (e % 4)
        return carry

    lax.fori_loop(0, K * 4 // 16, exp_body, 0)

    cnt_cap = jnp.minimum(cnt, K)
    nelem = cnt_cap * 4
    ng = (nelem + GW - 1) // GW

    def gath_body(g, carry):
        pltpu.async_copy(
            rpn_hbm.at[idx4_v.at[pl.ds(g * GW, GW)]],
            rows_v.at[pl.ds(g * GW, GW)],
            sem,
        ).wait()
        return carry

    lax.fori_loop(0, ng, gath_body, 0)

    def loss_body(t, acc):
        e = t * 16 + iota
        r = e // 4
        d = e % 4
        g_rank = jnp.minimum(base + r, MP - 1)
        tv = plsc.load_gather(tgt_v, [g_rank * 4 + d])
        pv = rows_v[pl.ds(t * 16, 16)]
        diff = jnp.abs(tv - pv)
        l = jnp.where(diff < 1.0, 0.5 * diff * diff, diff - 0.5)
        return acc + jnp.where(e < nelem, l, 0.0)

    nt = (nelem + 15) // 16
    acc = lax.fori_loop(0, nt, loss_body, jnp.zeros((16,), jnp.float32))
    part = jnp.sum(acc)

    sum_v[...] = jnp.where(
        iota == 0, part,
        jnp.where(iota == 1, cnt.astype(jnp.float32), jnp.float32(0)),
    )
    w = c * NS + s
    pltpu.sync_copy(sum_v, out_hbm.at[pl.ds(w * 16, 16)])


@jax.jit
def kernel(target_bbox, rpn_match, rpn_bbox):
    matchf = rpn_match.reshape(B * A)
    tgtf = target_bbox.reshape(B * MP * 4)
    rpnf = rpn_bbox.reshape(B * A * 4)
    r = _rpn_loss_sc(matchf, tgtf, rpnf).reshape(NW, 16)
    total = jnp.sum(r[:, 0])
    cntf = jnp.sum(r[:, 1])
    return total / (cntf * 4.0)


# speedup vs baseline: 137.1576x; 17.7007x over previous
import functools

import jax
import jax.numpy as jnp
from jax import lax
from jax.experimental import pallas as pl
from jax.experimental.pallas import tpu as pltpu
from jax.experimental.pallas import tpu_sc as plsc

B = 8
A = 261888
MP = 256
NQ = 4
CH = A // NQ
NV = CH // 16
U = 6
K = 512
GW = 128
NC = 2
NS = 16
NW = NC * NS

_mesh = plsc.VectorSubcoreMesh(
    core_axis_name="c", subcore_axis_name="s", num_cores=NC, num_subcores=NS
)


@functools.partial(
    pl.kernel,
    out_type=jax.ShapeDtypeStruct((NW * 16,), jnp.float32),
    mesh=_mesh,
    scratch_types=[
        pltpu.VMEM((CH,), jnp.int32),
        pltpu.VMEM((K,), jnp.int32),
        pltpu.VMEM((K * 4,), jnp.int32),
        pltpu.VMEM((K * 4,), jnp.float32),
        pltpu.VMEM((MP * 4,), jnp.float32),
        pltpu.VMEM((16,), jnp.float32),
        pltpu.SMEM((32,), jnp.int32),
        pltpu.SemaphoreType.DMA,
    ],
    compiler_params=pltpu.CompilerParams(needs_layout_passes=False),
)
def _rpn_loss_sc(match_hbm, tgt_hbm, rpn_hbm, out_hbm,
                 chunk_v, idx_v, idx4_v, rows_v, tgt_v, sum_v,
                 flag_smem, sem):
    c = lax.axis_index("c")
    s = lax.axis_index("s")
    b = c * 4 + s // 4
    q = s % 4
    img = s // 4
    iota = lax.iota(jnp.int32, 16)

    for w in range(18):
        flag_smem[w] = 0

    pltpu.sync_copy(match_hbm.at[pl.ds(b * A + q * CH, CH)], chunk_v)
    pltpu.sync_copy(tgt_hbm.at[pl.ds(b * (MP * 4), MP * 4)], tgt_v)

    row_base = b * A + q * CH

    def fill_body(i, carry):
        ag0 = q * CH + i * 16 + iota
        idx_v[pl.ds(i * 16, 16)] = b * (A * 4) + ((ag0 >> 7) << 9) + (ag0 & 127)
        return carry

    lax.fori_loop(0, K // 16, fill_body, 0)

    zimg = b * (A * 4)

    def scan_body(i, off_vec):
        for u in range(U):
            base_e = (i * U + u) * 16
            v = chunk_v[pl.ds(base_e, 16)]
            m = v == 1
            slot = off_vec + plsc.cumsum(m.astype(jnp.int32)) - 1
            slot = jnp.clip(slot, 0, K - 1)
            ag = q * CH + base_e + iota
            zrow = zimg + ((ag >> 7) << 9) + (ag & 127)
            plsc.store_scatter(idx_v, [slot], zrow, mask=m)
            off_vec = off_vec + plsc.all_reduce_population_count(m)
        return off_vec

    off_vec = lax.fori_loop(0, NV // U, scan_body, jnp.zeros((16,), jnp.int32))
    cnt = jnp.max(off_vec)

    for qq in range(NQ):
        add = jnp.where(qq > q, cnt, jnp.int32(0))
        plsc.fetch_and_add(
            flag_smem.at[2 + img * 4 + qq], add, subcore_id=jnp.int32(0)
        )
    plsc.fetch_and_add(flag_smem.at[0], jnp.int32(1), subcore_id=jnp.int32(0))

    def spin_cond(seen):
        return seen < NS

    def spin_body(seen):
        return plsc.fetch_and_add(
            flag_smem.at[0], jnp.int32(0), subcore_id=jnp.int32(0)
        )

    lax.while_loop(spin_cond, spin_body, jnp.int32(0))

    base = plsc.fetch_and_add(
        flag_smem.at[2 + img * 4 + q], jnp.int32(0), subcore_id=jnp.int32(0)
    )

    def exp_body(t, carry):
        e = t * 16 + iota
        flat = plsc.load_gather(idx_v, [e // 4])
        idx4_v[pl.ds(t * 16, 16)] = flat + (e % 4) * 128
        return carry

    lax.fori_loop(0, K * 4 // 16, exp_body, 0)

    cnt_cap = jnp.minimum(cnt, K)
    nelem = cnt_cap * 4
    ng = (nelem + GW - 1) // GW

    def gath_body(g, carry):
        pltpu.async_copy(
            rpn_hbm.at[idx4_v.at[pl.ds(g * GW, GW)]],
            rows_v.at[pl.ds(g * GW, GW)],
            sem,
        ).wait()
        return carry

    lax.fori_loop(0, ng, gath_body, 0)

    def loss_body(t, acc):
        e = t * 16 + iota
        r = e // 4
        d = e % 4
        g_rank = jnp.minimum(base + r, MP - 1)
        tv = plsc.load_gather(tgt_v, [g_rank * 4 + d])
        pv = rows_v[pl.ds(t * 16, 16)]
        diff = jnp.abs(tv - pv)
        l = jnp.where(diff < 1.0, 0.5 * diff * diff, diff - 0.5)
        return acc + jnp.where(e < nelem, l, 0.0)

    nt = (nelem + 15) // 16
    acc = lax.fori_loop(0, nt, loss_body, jnp.zeros((16,), jnp.float32))
    part = jnp.sum(acc)

    sum_v[...] = jnp.where(
        iota == 0, part,
        jnp.where(iota == 1, cnt.astype(jnp.float32), jnp.float32(0)),
    )
    w = c * NS + s
    pltpu.sync_copy(sum_v, out_hbm.at[pl.ds(w * 16, 16)])


@jax.jit
def kernel(target_bbox, rpn_match, rpn_bbox):
    matchf = rpn_match.reshape(B * A)
    tgtf = target_bbox.reshape(B * MP * 4)
    rpnf = (
        rpn_bbox.transpose(0, 2, 1)
        .reshape(B, 4, A // 128, 128)
        .transpose(0, 2, 1, 3)
        .reshape(B * A * 4)
    )
    r = _rpn_loss_sc(matchf, tgtf, rpnf).reshape(NW, 16)
    total = jnp.sum(r[:, 0])
    cntf = jnp.sum(r[:, 1])
    return total / (cntf * 4.0)


# speedup vs baseline: 168.2667x; 1.2268x over previous
import functools

import jax
import jax.numpy as jnp
from jax import lax
from jax.experimental import pallas as pl
from jax.experimental.pallas import tpu as pltpu
from jax.experimental.pallas import tpu_sc as plsc

B = 8
A = 261888
MP = 256
NQ = 4
CH = A // NQ
NV = CH // 16
U = 6
K = 512
GW = 128
NC = 2
NS = 16
NW = NC * NS

_mesh = plsc.VectorSubcoreMesh(
    core_axis_name="c", subcore_axis_name="s", num_cores=NC, num_subcores=NS
)


@functools.partial(
    pl.kernel,
    out_type=jax.ShapeDtypeStruct((NW * 16,), jnp.float32),
    mesh=_mesh,
    scratch_types=[
        pltpu.VMEM((CH,), jnp.int32),
        pltpu.VMEM((K,), jnp.int32),
        pltpu.VMEM((K * 4,), jnp.int32),
        pltpu.VMEM((K * 4,), jnp.float32),
        pltpu.VMEM((MP * 4,), jnp.float32),
        pltpu.VMEM((16,), jnp.float32),
        pltpu.SMEM((32,), jnp.int32),
        pltpu.SemaphoreType.DMA,
    ],
    compiler_params=pltpu.CompilerParams(needs_layout_passes=False),
)
def _rpn_loss_sc(match_hbm, tgt_hbm, rpn_hbm, out_hbm,
                 chunk_v, idx_v, idx4_v, rows_v, tgt_v, sum_v,
                 flag_smem, sem):
    c = lax.axis_index("c")
    s = lax.axis_index("s")
    b = c * 4 + s // 4
    q = s % 4
    img = s // 4
    iota = lax.iota(jnp.int32, 16)

    for w in range(18):
        flag_smem[w] = 0

    pltpu.sync_copy(match_hbm.at[pl.ds(b * A + q * CH, CH)], chunk_v)
    pltpu.sync_copy(tgt_hbm.at[pl.ds(b * (MP * 4), MP * 4)], tgt_v)

    row_base = b * A + q * CH

    def fill_body(i, carry):
        ag0 = q * CH + i * 16 + iota
        idx_v[pl.ds(i * 16, 16)] = b * (A * 4) + ((ag0 >> 7) << 9) + (ag0 & 127)
        return carry

    lax.fori_loop(0, K // 16, fill_body, 0)

    zimg = b * (A * 4)

    def scan_slow(i, off_vec):
        for u in range(U):
            base_e = (i * U + u) * 16
            v = chunk_v[pl.ds(base_e, 16)]
            m = v == 1
            slot = off_vec + plsc.cumsum(m.astype(jnp.int32)) - 1
            slot = jnp.clip(slot, 0, K - 1)
            ag = q * CH + base_e + iota
            zrow = zimg + ((ag >> 7) << 9) + (ag & 127)
            plsc.store_scatter(idx_v, [slot], zrow, mask=m)
            off_vec = off_vec + plsc.all_reduce_population_count(m)
        return off_vec

    def scan_body(i, off_vec):
        z = chunk_v[pl.ds(i * U * 16, 16)] ^ 1
        for u in range(1, U):
            z = jnp.minimum(z, chunk_v[pl.ds((i * U + u) * 16, 16)] ^ 1)
        return lax.cond(
            jnp.any(z == 0), lambda: scan_slow(i, off_vec), lambda: off_vec
        )

    off_vec = lax.fori_loop(0, NV // U, scan_body, jnp.zeros((16,), jnp.int32))
    cnt = jnp.max(off_vec)

    for qq in range(NQ):
        add = jnp.where(qq > q, cnt, jnp.int32(0))
        plsc.fetch_and_add(
            flag_smem.at[2 + img * 4 + qq], add, subcore_id=jnp.int32(0)
        )
    plsc.fetch_and_add(flag_smem.at[0], jnp.int32(1), subcore_id=jnp.int32(0))

    def spin_cond(seen):
        return seen < NS

    def spin_body(seen):
        return plsc.fetch_and_add(
            flag_smem.at[0], jnp.int32(0), subcore_id=jnp.int32(0)
        )

    lax.while_loop(spin_cond, spin_body, jnp.int32(0))

    base = plsc.fetch_and_add(
        flag_smem.at[2 + img * 4 + q], jnp.int32(0), subcore_id=jnp.int32(0)
    )

    def exp_body(t, carry):
        e = t * 16 + iota
        flat = plsc.load_gather(idx_v, [e // 4])
        idx4_v[pl.ds(t * 16, 16)] = flat + (e % 4) * 128
        return carry

    lax.fori_loop(0, K * 4 // 16, exp_body, 0)

    cnt_cap = jnp.minimum(cnt, K)
    nelem = cnt_cap * 4
    ng = (nelem + GW - 1) // GW

    def gath_body(g, carry):
        pltpu.async_copy(
            rpn_hbm.at[idx4_v.at[pl.ds(g * GW, GW)]],
            rows_v.at[pl.ds(g * GW, GW)],
            sem,
        ).wait()
        return carry

    lax.fori_loop(0, ng, gath_body, 0)

    def loss_body(t, acc):
        e = t * 16 + iota
        r = e // 4
        d = e % 4
        g_rank = jnp.minimum(base + r, MP - 1)
        tv = plsc.load_gather(
            tgt_v, [((g_rank >> 7) << 9) + d * 128 + (g_rank & 127)]
        )
        pv = rows_v[pl.ds(t * 16, 16)]
        diff = jnp.abs(tv - pv)
        l = jnp.where(diff < 1.0, 0.5 * diff * diff, diff - 0.5)
        return acc + jnp.where(e < nelem, l, 0.0)

    nt = (nelem + 15) // 16
    acc = lax.fori_loop(0, nt, loss_body, jnp.zeros((16,), jnp.float32))
    part = jnp.sum(acc)

    sum_v[...] = jnp.where(
        iota == 0, part,
        jnp.where(iota == 1, cnt.astype(jnp.float32), jnp.float32(0)),
    )
    w = c * NS + s
    pltpu.sync_copy(sum_v, out_hbm.at[pl.ds(w * 16, 16)])


@jax.jit
def kernel(target_bbox, rpn_match, rpn_bbox):
    matchf = rpn_match.reshape(B * A)
    tgtf = (
        target_bbox.transpose(0, 2, 1)
        .reshape(B, 4, MP // 128, 128)
        .transpose(0, 2, 1, 3)
        .reshape(B * MP * 4)
    )
    rpnf = (
        rpn_bbox.transpose(0, 2, 1)
        .reshape(B, 4, A // 128, 128)
        .transpose(0, 2, 1, 3)
        .reshape(B * A * 4)
    )
    r = _rpn_loss_sc(matchf, tgtf, rpnf).reshape(NW, 16)
    total = jnp.sum(r[:, 0])
    cntf = jnp.sum(r[:, 1])
    return total / (cntf * 4.0)
